# baseline (speedup 1.0000x reference)
import jax
import jax.numpy as jnp
from jax.experimental import pallas as pl
from jax.experimental.pallas import tpu as pltpu

_FC = 2048
_BN_EPS = 1e-5
_TB = 512


def _mlp_kernel(z_ref, w1_ref, s1_ref, t1_ref, w2_ref, s2_ref, t2_ref,
                w3_ref, b3_ref, o_ref):
    h = jnp.dot(z_ref[...], w1_ref[...], preferred_element_type=jnp.float32)
    h = h * s1_ref[...] + t1_ref[...]
    h = jnp.where(h >= 0, h, 0.02 * h).astype(jnp.bfloat16)

    h = jnp.dot(h, w2_ref[...], preferred_element_type=jnp.float32)
    h = h * s2_ref[...] + t2_ref[...]
    h = jnp.where(h >= 0, h, 0.02 * h).astype(jnp.bfloat16)

    y = jnp.dot(h, w3_ref[...], preferred_element_type=jnp.float32)
    o_ref[...] = y + b3_ref[...]


def kernel(z, l1_w, l1_b, bn1_g, bn1_b, bn1_m, bn1_v,
           l2_w, l2_b, bn2_g, bn2_b, bn2_m, bn2_v, l3_w, l3_b):
    B = 2048
    z = z.reshape(B, -1)
    nz = z.shape[1]
    no = l3_w.shape[1]

    s1 = bn1_g * jax.lax.rsqrt(bn1_v + _BN_EPS)
    t1 = bn1_b + (l1_b - bn1_m) * s1
    s2 = bn2_g * jax.lax.rsqrt(bn2_v + _BN_EPS)
    t2 = bn2_b + (l2_b - bn2_m) * s2

    zb = z.astype(jnp.bfloat16)
    w1 = l1_w.astype(jnp.bfloat16)
    w2 = l2_w.astype(jnp.bfloat16)
    w3 = l3_w.astype(jnp.bfloat16)

    return pl.pallas_call(
        _mlp_kernel,
        out_shape=jax.ShapeDtypeStruct((B, no), jnp.float32),
        grid=(B // _TB,),
        in_specs=[
            pl.BlockSpec((_TB, nz), lambda i: (i, 0)),
            pl.BlockSpec((nz, _FC), lambda i: (0, 0)),
            pl.BlockSpec((1, _FC), lambda i: (0, 0)),
            pl.BlockSpec((1, _FC), lambda i: (0, 0)),
            pl.BlockSpec((_FC, _FC), lambda i: (0, 0)),
            pl.BlockSpec((1, _FC), lambda i: (0, 0)),
            pl.BlockSpec((1, _FC), lambda i: (0, 0)),
            pl.BlockSpec((_FC, no), lambda i: (0, 0)),
            pl.BlockSpec((1, no), lambda i: (0, 0)),
        ],
        out_specs=pl.BlockSpec((_TB, no), lambda i: (i, 0)),
        compiler_params=pltpu.CompilerParams(dimension_semantics=("parallel",)),
    )(zb, w1, s1.reshape(1, _FC), t1.reshape(1, _FC),
      w2, s2.reshape(1, _FC), t2.reshape(1, _FC),
      w3, l3_b.reshape(1, no))


# speedup vs baseline: 1.1979x; 1.1979x over previous
import jax
import jax.numpy as jnp
from jax.experimental import pallas as pl
from jax.experimental.pallas import tpu as pltpu

_FC = 2048
_NO = 3072
_B = 2048
_BN_EPS = 1e-5
_TB = 512
_CR = 256


def _mlp_kernel(z_ref, w1_ref, b1_ref, g1_ref, be1_ref, m1_ref, v1_ref,
                b2_ref, g2_ref, be2_ref, m2_ref, v2_ref, b3_ref,
                w2_hbm, w3_hbm, o_ref,
                w2b, w3b, stage2, stage3, sem2, sem3):
    i = pl.program_id(0)

    @pl.when(i == 0)
    def _load_weights():
        n2 = _FC // _CR
        n3 = _FC // _CR
        n = n2 + n3

        def start(k):
            s = k % 2
            if k < n2:
                pltpu.make_async_copy(
                    w2_hbm.at[pl.ds(k * _CR, _CR), :],
                    stage2.at[s], sem2.at[s]).start()
            else:
                pltpu.make_async_copy(
                    w3_hbm.at[pl.ds((k - n2) * _CR, _CR), :],
                    stage3.at[s], sem3.at[s]).start()

        start(0)
        start(1)
        for k in range(n):
            s = k % 2
            if k < n2:
                pltpu.make_async_copy(stage2.at[s], stage2.at[s],
                                      sem2.at[s]).wait()
                w2b[pl.ds(k * _CR, _CR), :] = stage2[s].astype(jnp.bfloat16)
            else:
                pltpu.make_async_copy(stage3.at[s], stage3.at[s],
                                      sem3.at[s]).wait()
                w3b[pl.ds((k - n2) * _CR, _CR), :] = stage3[s].astype(jnp.bfloat16)
            if k + 2 < n:
                start(k + 2)

    s1 = g1_ref[...] * jax.lax.rsqrt(v1_ref[...] + _BN_EPS)
    t1 = be1_ref[...] + (b1_ref[...] - m1_ref[...]) * s1
    s2 = g2_ref[...] * jax.lax.rsqrt(v2_ref[...] + _BN_EPS)
    t2 = be2_ref[...] + (b2_ref[...] - m2_ref[...]) * s2

    zb = z_ref[...].astype(jnp.bfloat16)
    w1 = w1_ref[...].astype(jnp.bfloat16)

    h = jnp.dot(zb, w1, preferred_element_type=jnp.float32)
    h = h * s1 + t1
    h = jnp.where(h >= 0, h, 0.02 * h).astype(jnp.bfloat16)

    h = jnp.dot(h, w2b[...], preferred_element_type=jnp.float32)
    h = h * s2 + t2
    h = jnp.where(h >= 0, h, 0.02 * h).astype(jnp.bfloat16)

    y = jnp.dot(h, w3b[...], preferred_element_type=jnp.float32)
    o_ref[...] = y + b3_ref[...]


def kernel(z, l1_w, l1_b, bn1_g, bn1_b, bn1_m, bn1_v,
           l2_w, l2_b, bn2_g, bn2_b, bn2_m, bn2_v, l3_w, l3_b):
    z = z.reshape(_B, -1)
    nz = z.shape[1]

    vec = lambda a: a.reshape(1, -1)
    const = lambda shape: pl.BlockSpec(shape, lambda i: (0, 0))

    return pl.pallas_call(
        _mlp_kernel,
        out_shape=jax.ShapeDtypeStruct((_B, _NO), jnp.float32),
        grid=(_B // _TB,),
        in_specs=[
            pl.BlockSpec((_TB, nz), lambda i: (i, 0)),
            const((nz, _FC)),
            const((1, _FC)), const((1, _FC)), const((1, _FC)),
            const((1, _FC)), const((1, _FC)),
            const((1, _FC)), const((1, _FC)), const((1, _FC)),
            const((1, _FC)), const((1, _FC)),
            const((1, _NO)),
            pl.BlockSpec(memory_space=pl.ANY),
            pl.BlockSpec(memory_space=pl.ANY),
        ],
        out_specs=pl.BlockSpec((_TB, _NO), lambda i: (i, 0)),
        scratch_shapes=[
            pltpu.VMEM((_FC, _FC), jnp.bfloat16),
            pltpu.VMEM((_FC, _NO), jnp.bfloat16),
            pltpu.VMEM((2, _CR, _FC), jnp.float32),
            pltpu.VMEM((2, _CR, _NO), jnp.float32),
            pltpu.SemaphoreType.DMA((2,)),
            pltpu.SemaphoreType.DMA((2,)),
        ],
        compiler_params=pltpu.CompilerParams(
            dimension_semantics=("arbitrary",)),
    )(z, l1_w,
      vec(l1_b), vec(bn1_g), vec(bn1_b), vec(bn1_m), vec(bn1_v),
      vec(l2_b), vec(bn2_g), vec(bn2_b), vec(bn2_m), vec(bn2_v),
      vec(l3_b), l2_w, l3_w)


# speedup vs baseline: 1.1997x; 1.0015x over previous
import jax
import jax.numpy as jnp
from jax.experimental import pallas as pl
from jax.experimental.pallas import tpu as pltpu

_FC = 2048
_NO = 3072
_B = 2048
_BN_EPS = 1e-5
_TB = 512
_CC = 256
_NS = 3


def _leaky(x):
    return jnp.where(x >= 0, x, 0.02 * x)


def _mlp_kernel(z_ref, w1_ref, b1_ref, g1_ref, be1_ref, m1_ref, v1_ref,
                b2_ref, g2_ref, be2_ref, m2_ref, v2_ref, b3_ref,
                w2_hbm, w3_hbm, o_ref,
                w2b, w3b, h2b, stage, sem):
    i = pl.program_id(0)

    s1 = g1_ref[...] * jax.lax.rsqrt(v1_ref[...] + _BN_EPS)
    t1 = be1_ref[...] + (b1_ref[...] - m1_ref[...]) * s1
    s2 = g2_ref[...] * jax.lax.rsqrt(v2_ref[...] + _BN_EPS)
    t2 = be2_ref[...] + (b2_ref[...] - m2_ref[...]) * s2

    zb = z_ref[...].astype(jnp.bfloat16)
    w1 = w1_ref[...].astype(jnp.bfloat16)
    h1 = jnp.dot(zb, w1, preferred_element_type=jnp.float32)
    h1 = _leaky(h1 * s1 + t1).astype(jnp.bfloat16)

    n2 = _FC // _CC
    n3 = _NO // _CC

    @pl.when(i == 0)
    def _stream_and_compute():
        def start(k):
            s = k % _NS
            if k < n2:
                src = w2_hbm.at[:, pl.ds(k * _CC, _CC)]
            else:
                src = w3_hbm.at[:, pl.ds((k - n2) * _CC, _CC)]
            pltpu.make_async_copy(src, stage.at[s], sem.at[s]).start()

        for k in range(_NS):
            start(k)
        for k in range(n2):
            s = k % _NS
            pltpu.make_async_copy(stage.at[s], stage.at[s], sem.at[s]).wait()
            wc = stage[s].astype(jnp.bfloat16)
            w2b[:, pl.ds(k * _CC, _CC)] = wc
            if k + _NS < n2 + n3:
                start(k + _NS)
            hc = jnp.dot(h1, wc, preferred_element_type=jnp.float32)
            hc = hc * s2[:, k * _CC:(k + 1) * _CC] + t2[:, k * _CC:(k + 1) * _CC]
            h2b[:, pl.ds(k * _CC, _CC)] = _leaky(hc).astype(jnp.bfloat16)
        for k in range(n3):
            kk = k + n2
            s = kk % _NS
            pltpu.make_async_copy(stage.at[s], stage.at[s], sem.at[s]).wait()
            wc = stage[s].astype(jnp.bfloat16)
            w3b[:, pl.ds(k * _CC, _CC)] = wc
            if kk + _NS < n2 + n3:
                start(kk + _NS)
            y = jnp.dot(h2b[...], wc, preferred_element_type=jnp.float32)
            o_ref[:, pl.ds(k * _CC, _CC)] = y + b3_ref[:, k * _CC:(k + 1) * _CC]

    @pl.when(i > 0)
    def _steady():
        h2 = jnp.dot(h1, w2b[...], preferred_element_type=jnp.float32)
        h2 = _leaky(h2 * s2 + t2).astype(jnp.bfloat16)
        y = jnp.dot(h2, w3b[...], preferred_element_type=jnp.float32)
        o_ref[...] = y + b3_ref[...]


def kernel(z, l1_w, l1_b, bn1_g, bn1_b, bn1_m, bn1_v,
           l2_w, l2_b, bn2_g, bn2_b, bn2_m, bn2_v, l3_w, l3_b):
    z = z.reshape(_B, -1)
    nz = z.shape[1]

    vec = lambda a: a.reshape(1, -1)
    const = lambda shape: pl.BlockSpec(shape, lambda i: (0, 0))

    return pl.pallas_call(
        _mlp_kernel,
        out_shape=jax.ShapeDtypeStruct((_B, _NO), jnp.float32),
        grid=(_B // _TB,),
        in_specs=[
            pl.BlockSpec((_TB, nz), lambda i: (i, 0)),
            const((nz, _FC)),
            const((1, _FC)), const((1, _FC)), const((1, _FC)),
            const((1, _FC)), const((1, _FC)),
            const((1, _FC)), const((1, _FC)), const((1, _FC)),
            const((1, _FC)), const((1, _FC)),
            const((1, _NO)),
            pl.BlockSpec(memory_space=pl.ANY),
            pl.BlockSpec(memory_space=pl.ANY),
        ],
        out_specs=pl.BlockSpec((_TB, _NO), lambda i: (i, 0)),
        scratch_shapes=[
            pltpu.VMEM((_FC, _FC), jnp.bfloat16),
            pltpu.VMEM((_FC, _NO), jnp.bfloat16),
            pltpu.VMEM((_TB, _FC), jnp.bfloat16),
            pltpu.VMEM((_NS, _FC, _CC), jnp.float32),
            pltpu.SemaphoreType.DMA((_NS,)),
        ],
        compiler_params=pltpu.CompilerParams(
            dimension_semantics=("arbitrary",)),
    )(z, l1_w,
      vec(l1_b), vec(bn1_g), vec(bn1_b), vec(bn1_m), vec(bn1_v),
      vec(l2_b), vec(bn2_g), vec(bn2_b), vec(bn2_m), vec(bn2_v),
      vec(l3_b), l2_w, l3_w)
